# baseline (speedup 1.0000x reference)
import functools

import jax
import jax.numpy as jnp
from jax import lax
from jax.experimental import pallas as pl
from jax.experimental.pallas import tpu as pltpu
from jax.experimental.pallas import tpu_sc as plsc

MAX_H = 64
MAX_W = 64

NC = 2
NS = 16
L = 16
NRING = 4

TCB = 80
BB = 8


def _tc_part(x, pos_table):
    B, H, W, C = x.shape
    WC = W * C
    x_r = x.reshape(B, H, WC)
    pt_r = pos_table.reshape(MAX_H, MAX_W * C)

    def body(x_ref, pt_ref, o_ref):
        o_ref[...] = x_ref[...] + pt_ref[:H, :WC][None, :, :]

    out = pl.pallas_call(
        body,
        grid=(TCB // BB,),
        in_specs=[
            pl.BlockSpec((BB, H, WC), lambda i: (i, 0, 0)),
            pl.BlockSpec((MAX_H, MAX_W * C), lambda i: (0, 0)),
        ],
        out_specs=pl.BlockSpec((BB, H, WC), lambda i: (i, 0, 0)),
        out_shape=jax.ShapeDtypeStruct((TCB, H, WC), x.dtype),
    )(x_r, pt_r)
    return out.reshape(TCB, H, W, C)


def _make_sc_part(B, H, W, C):
    scb = B - TCB
    mesh = plsc.VectorSubcoreMesh(core_axis_name="c", subcore_axis_name="s")

    @functools.partial(
        pl.kernel,
        mesh=mesh,
        out_type=jax.ShapeDtypeStruct((scb, H, W, C), jnp.float32),
        scratch_types=[
            pltpu.VMEM((W, C), jnp.float32),
            pltpu.VMEM((NRING, W, C), jnp.float32),
            pltpu.VMEM((NRING, W, C), jnp.float32),
        ]
        + [pltpu.SemaphoreType.DMA] * (2 * NRING),
    )
    def sc_kernel(x_hbm, pt_hbm, o_hbm, posb, in_b, out_b, *sems):
        in_sems = sems[:NRING]
        out_sems = sems[NRING:]
        h = lax.axis_index("s") * NC + lax.axis_index("c")

        pltpu.sync_copy(pt_hbm.at[pl.ds(h * MAX_W, W)], posb)

        def start_in(b, slot):
            pltpu.make_async_copy(
                x_hbm.at[TCB + b, h], in_b.at[slot], in_sems[slot]
            ).start()

        for s in range(NRING):
            start_in(s, s)

        def add_slab(slot):
            @plsc.parallel_loop(0, W, unroll=8)
            def _(r):
                for j in range(C // L):
                    out_b[slot, r, pl.ds(j * L, L)] = (
                        in_b[slot, r, pl.ds(j * L, L)] + posb[r, pl.ds(j * L, L)]
                    )

        def group(g, carry):
            for s in range(NRING):
                b = g * NRING + s
                pltpu.make_async_copy(
                    x_hbm.at[TCB + b, h], in_b.at[s], in_sems[s]
                ).wait()

                @pl.when(g >= 1)
                def _():
                    pltpu.make_async_copy(
                        out_b.at[s], o_hbm.at[b - NRING, h], out_sems[s]
                    ).wait()

                add_slab(s)

                pltpu.make_async_copy(
                    out_b.at[s], o_hbm.at[b, h], out_sems[s]
                ).start()

                @pl.when(b + NRING < scb)
                def _():
                    start_in(b + NRING, s)

            return carry

        lax.fori_loop(0, scb // NRING, group, 0)

        for s in range(NRING):
            pltpu.make_async_copy(
                out_b.at[s], o_hbm.at[0, h], out_sems[s]
            ).wait()

    return sc_kernel


def kernel(x, pos_table):
    B, H, W, C = x.shape
    tc_out = _tc_part(x, pos_table)
    sc_out = _make_sc_part(B, H, W, C)(x, pos_table)
    return jnp.concatenate([tc_out, sc_out], axis=0)


# speedup vs baseline: 2.2478x; 2.2478x over previous
import jax
import jax.numpy as jnp
from jax.experimental import pallas as pl
from jax.experimental.pallas import tpu as pltpu

MAX_H = 64
MAX_W = 64

BB = 8


def kernel(x, pos_table):
    B, H, W, C = x.shape
    WC = W * C
    x_r = x.reshape(B, H, WC)
    pt_r = pos_table.reshape(MAX_H, MAX_W * C)

    def body(x_ref, pt_ref, o_ref):
        o_ref[...] = x_ref[...] + pt_ref[:H, :WC][None, :, :]

    out = pl.pallas_call(
        body,
        grid=(B // BB,),
        in_specs=[
            pl.BlockSpec((BB, H, WC), lambda i: (i, 0, 0)),
            pl.BlockSpec((MAX_H, MAX_W * C), lambda i: (0, 0)),
        ],
        out_specs=pl.BlockSpec((BB, H, WC), lambda i: (i, 0, 0)),
        out_shape=jax.ShapeDtypeStruct((B, H, WC), x.dtype),
    )(x_r, pt_r)
    return out.reshape(B, H, W, C)


# speedup vs baseline: 2.2659x; 1.0081x over previous
import jax
import jax.numpy as jnp
from jax.experimental import pallas as pl
from jax.experimental.pallas import tpu as pltpu

MAX_H = 64
MAX_W = 64

BB = 16


def kernel(x, pos_table):
    B, H, W, C = x.shape
    WC = W * C
    x_r = x.reshape(B, H, WC)
    pt_r = pos_table.reshape(MAX_H, MAX_W * C)

    def body(x_ref, pt_ref, o_ref):
        o_ref[...] = x_ref[...] + pt_ref[:H, :WC][None, :, :]

    out = pl.pallas_call(
        body,
        grid=(B // BB,),
        in_specs=[
            pl.BlockSpec((BB, H, WC), lambda i: (i, 0, 0)),
            pl.BlockSpec((MAX_H, MAX_W * C), lambda i: (0, 0)),
        ],
        out_specs=pl.BlockSpec((BB, H, WC), lambda i: (i, 0, 0)),
        out_shape=jax.ShapeDtypeStruct((B, H, WC), x.dtype),
    )(x_r, pt_r)
    return out.reshape(B, H, W, C)


# speedup vs baseline: 2.2962x; 1.0134x over previous
import jax
import jax.numpy as jnp
from jax.experimental import pallas as pl
from jax.experimental.pallas import tpu as pltpu

MAX_H = 64
MAX_W = 64

BB = 32


def kernel(x, pos_table):
    B, H, W, C = x.shape
    WC = W * C
    x_r = x.reshape(B, H, WC)
    pt_r = pos_table.reshape(MAX_H, MAX_W * C)

    def body(x_ref, pt_ref, o_ref):
        o_ref[...] = x_ref[...] + pt_ref[:H, :WC][None, :, :]

    out = pl.pallas_call(
        body,
        grid=(B // BB,),
        in_specs=[
            pl.BlockSpec((BB, H, WC), lambda i: (i, 0, 0)),
            pl.BlockSpec((MAX_H, MAX_W * C), lambda i: (0, 0)),
        ],
        out_specs=pl.BlockSpec((BB, H, WC), lambda i: (i, 0, 0)),
        out_shape=jax.ShapeDtypeStruct((B, H, WC), x.dtype),
    )(x_r, pt_r)
    return out.reshape(B, H, W, C)
